# initial kernel scaffold (unmeasured)
import jax
import jax.numpy as jnp
from jax import lax
from jax.experimental import pallas as pl
from jax.experimental.pallas import tpu as pltpu

N_DEV = 4
NT = 4


def kernel(x, w_mat):
    m_per, k = x.shape
    _, n = w_mat.shape
    n_per = n // N_DEV
    tile = n_per // NT

    x = x.astype(jnp.bfloat16)

    ORDER = [1, 3, 2, 0]

    def body(x_ref, w_ref, out_ref, w_buf, send_buf, recv_buf,
             w_sems, send_sems, recv_sems):
        my = lax.axis_index("i")

        steps = [(d, t) for d in ORDER for t in range(NT)]

        def w_tile_copy(d, t, slot):
            dst = (my + d) % N_DEV
            return pltpu.make_async_copy(
                w_ref.at[:, pl.ds(dst * n_per + t * tile, tile)],
                w_buf.at[slot],
                w_sems.at[slot],
            )

        w_tile_copy(*steps[0], 0).start()

        for s, (d, t) in enumerate(steps):
            slot = s % 2
            if s + 1 < len(steps):
                w_tile_copy(*steps[s + 1], (s + 1) % 2).start()
            w_tile_copy(d, t, slot).wait()

            wt = w_buf[slot].astype(jnp.bfloat16)
            yt = jnp.maximum(
                jnp.dot(x_ref[...], wt, preferred_element_type=jnp.float32),
                0.0,
            )
            if d == 0:
                out_ref[pl.ds(my * m_per, m_per), pl.ds(t * tile, tile)] = yt
            else:
                send_buf[d - 1, :, pl.ds(t * tile, tile)] = yt.astype(jnp.bfloat16)
                if t == NT - 1:
                    rdma = pltpu.make_async_remote_copy(
                        src_ref=send_buf.at[d - 1],
                        dst_ref=recv_buf.at[d - 1],
                        send_sem=send_sems.at[d - 1],
                        recv_sem=recv_sems.at[d - 1],
                        device_id=((my + d) % N_DEV,),
                        device_id_type=pl.DeviceIdType.MESH,
                    )
                    rdma.start()

        for d in [1, 3, 2]:
            desc = pltpu.make_async_remote_copy(
                src_ref=send_buf.at[d - 1],
                dst_ref=recv_buf.at[d - 1],
                send_sem=send_sems.at[d - 1],
                recv_sem=recv_sems.at[d - 1],
                device_id=((my + d) % N_DEV,),
                device_id_type=pl.DeviceIdType.MESH,
            )
            desc.wait_recv()
            src = (my - d) % N_DEV
            out_ref[pl.ds(src * m_per, m_per), :] = recv_buf[d - 1].astype(
                jnp.float32
            )
        for d in [1, 3, 2]:
            desc = pltpu.make_async_remote_copy(
                src_ref=send_buf.at[d - 1],
                dst_ref=recv_buf.at[d - 1],
                send_sem=send_sems.at[d - 1],
                recv_sem=recv_sems.at[d - 1],
                device_id=((my + d) % N_DEV,),
                device_id_type=pl.DeviceIdType.MESH,
            )
            desc.wait_send()

    return pl.pallas_call(
        body,
        out_shape=jax.ShapeDtypeStruct((N_DEV * m_per, n_per), jnp.float32),
        in_specs=[
            pl.BlockSpec(memory_space=pltpu.VMEM),
            pl.BlockSpec(memory_space=pltpu.ANY),
        ],
        out_specs=pl.BlockSpec(memory_space=pltpu.VMEM),
        scratch_shapes=[
            pltpu.VMEM((2, k, tile), jnp.float32),
            pltpu.VMEM((N_DEV - 1, m_per, n_per), jnp.bfloat16),
            pltpu.VMEM((N_DEV - 1, m_per, n_per), jnp.bfloat16),
            pltpu.SemaphoreType.DMA((2,)),
            pltpu.SemaphoreType.DMA((N_DEV - 1,)),
            pltpu.SemaphoreType.DMA((N_DEV - 1,)),
        ],
    )(x, w_mat)


# baseline (device time: 213744 ns/iter reference)
import jax
import jax.numpy as jnp
from jax import lax
from jax.experimental import pallas as pl
from jax.experimental.pallas import tpu as pltpu

N_DEV = 4
NT = 8


def kernel(x, w_mat):
    m_per, k = x.shape
    _, n = w_mat.shape
    n_per = n // N_DEV
    tile = n_per // NT

    x = x.astype(jnp.bfloat16)

    ORDER = [1, 3, 2, 0]

    def body(x_ref, w_ref, out_ref, w_buf, send_buf, recv_buf, stage,
             w_sems, send_sems, recv_sems, out_sem):
        my = lax.axis_index("i")

        steps = [(d, t) for d in ORDER for t in range(NT)]

        def w_tile_copy(d, t, slot):
            dst = (my + d) % N_DEV
            return pltpu.make_async_copy(
                w_ref.at[:, pl.ds(dst * n_per + t * tile, tile)],
                w_buf.at[slot],
                w_sems.at[slot],
            )

        def remote_desc(d):
            return pltpu.make_async_remote_copy(
                src_ref=send_buf.at[d - 1],
                dst_ref=recv_buf.at[d - 1],
                send_sem=send_sems.at[d - 1],
                recv_sem=recv_sems.at[d - 1],
                device_id=((my + d) % N_DEV,),
                device_id_type=pl.DeviceIdType.MESH,
            )

        def stage_out_copy(src_rows):
            return pltpu.make_async_copy(
                stage,
                out_ref.at[pl.ds(src_rows * m_per, m_per), :],
                out_sem,
            )

        w_tile_copy(*steps[0], 0).start()

        for s, (d, t) in enumerate(steps):
            slot = s % 2
            if s + 1 < len(steps):
                w_tile_copy(*steps[s + 1], (s + 1) % 2).start()
            w_tile_copy(d, t, slot).wait()

            wt = w_buf[slot].astype(jnp.bfloat16)
            yt = jnp.maximum(
                jnp.dot(x_ref[...], wt, preferred_element_type=jnp.float32),
                0.0,
            )
            if d == 0:
                stage[:, pl.ds(t * tile, tile)] = yt
                if t == NT - 1:
                    stage_out_copy(my).start()
            else:
                send_buf[d - 1, :, pl.ds(t * tile, tile)] = yt.astype(jnp.bfloat16)
                if t == NT - 1:
                    remote_desc(d).start()

        stage_out_copy(my).wait()
        for d in [1, 3, 2]:
            remote_desc(d).wait_recv()
            src = (my - d) % N_DEV
            stage[...] = recv_buf[d - 1].astype(jnp.float32)
            stage_out_copy(src).start()
            stage_out_copy(src).wait()
        for d in [1, 3, 2]:
            remote_desc(d).wait_send()

    return pl.pallas_call(
        body,
        out_shape=jax.ShapeDtypeStruct((N_DEV * m_per, n_per), jnp.float32),
        in_specs=[
            pl.BlockSpec(memory_space=pltpu.MemorySpace.VMEM),
            pl.BlockSpec(memory_space=pl.ANY),
        ],
        out_specs=pl.BlockSpec(memory_space=pl.ANY),
        scratch_shapes=[
            pltpu.VMEM((2, k, tile), jnp.float32),
            pltpu.VMEM((N_DEV - 1, m_per, n_per), jnp.bfloat16),
            pltpu.VMEM((N_DEV - 1, m_per, n_per), jnp.bfloat16),
            pltpu.VMEM((m_per, n_per), jnp.float32),
            pltpu.SemaphoreType.DMA((2,)),
            pltpu.SemaphoreType.DMA((N_DEV - 1,)),
            pltpu.SemaphoreType.DMA((N_DEV - 1,)),
            pltpu.SemaphoreType.DMA,
        ],
        compiler_params=pltpu.CompilerParams(
            vmem_limit_bytes=62 * 1024 * 1024,
        ),
    )(x, w_mat)


# device time: 153560 ns/iter; 1.3919x vs baseline; 1.3919x over previous
import jax
import jax.numpy as jnp
from jax import lax
from jax.experimental import pallas as pl
from jax.experimental.pallas import tpu as pltpu

N_DEV = 4
NT = 4
ORDER = [2, 1, 3, 0]


def kernel(x, w_mat):
    m_per, k = x.shape
    _, n = w_mat.shape
    n_per = n // N_DEV
    tile = n_per // NT
    half = n_per // 2

    x = x.astype(jnp.bfloat16)

    def body(x_ref, w_ref, out_ref, w_buf, send_buf, recv_buf, stage,
             w_sems, send_sems, recv_sems, out_sems):
        my = lax.axis_index("i")

        steps = [(d, t) for d in ORDER for t in range(NT)]

        def w_tile_copy(d, t, slot):
            dst = (my + d) % N_DEV
            return pltpu.make_async_copy(
                w_ref.at[:, pl.ds(dst * n_per + t * tile, tile)],
                w_buf.at[slot],
                w_sems.at[slot],
            )

        def remote_desc(d, h):
            hs = pl.ds(h * half, half)
            return pltpu.make_async_remote_copy(
                src_ref=send_buf.at[d - 1, :, hs],
                dst_ref=recv_buf.at[d - 1, :, hs],
                send_sem=send_sems.at[d - 1, h],
                recv_sem=recv_sems.at[d - 1, h],
                device_id=((my + d) % N_DEV,),
                device_id_type=pl.DeviceIdType.MESH,
            )

        def out_copy(src_rows, h):
            hs = pl.ds(h * half, half)
            return pltpu.make_async_copy(
                stage.at[:, hs],
                out_ref.at[pl.ds(src_rows * m_per, m_per), hs],
                out_sems.at[h],
            )

        w_tile_copy(*steps[0], 0).start()

        for s, (d, t) in enumerate(steps):
            slot = s % 2
            if s + 1 < len(steps):
                w_tile_copy(*steps[s + 1], (s + 1) % 2).start()
            w_tile_copy(d, t, slot).wait()

            wt = w_buf[slot].astype(jnp.bfloat16)
            yt = jnp.maximum(
                jnp.dot(x_ref[...], wt, preferred_element_type=jnp.float32),
                0.0,
            )
            if d == 0:
                stage[:, pl.ds(t * tile, tile)] = yt
                if t * tile + tile in (half, n_per):
                    out_copy(my, (t * tile) // half).start()
            else:
                send_buf[d - 1, :, pl.ds(t * tile, tile)] = yt.astype(jnp.bfloat16)
                if t * tile + tile in (half, n_per):
                    remote_desc(d, (t * tile) // half).start()

        for d in [1, 3, 2]:
            src = (my - d) % N_DEV
            for h in (0, 1):
                hs = pl.ds(h * half, half)
                remote_desc(d, h).wait_recv()
                out_copy(my, h).wait()
                stage[:, hs] = recv_buf[d - 1, :, hs].astype(jnp.float32)
                out_copy(src, h).start()
        for h in (0, 1):
            out_copy(my, h).wait()
        for d in [1, 3, 2]:
            for h in (0, 1):
                remote_desc(d, h).wait_send()

    return pl.pallas_call(
        body,
        out_shape=jax.ShapeDtypeStruct((N_DEV * m_per, n_per), jnp.float32),
        in_specs=[
            pl.BlockSpec(memory_space=pltpu.MemorySpace.VMEM),
            pl.BlockSpec(memory_space=pl.ANY),
        ],
        out_specs=pl.BlockSpec(memory_space=pl.ANY),
        scratch_shapes=[
            pltpu.VMEM((2, k, tile), jnp.float32),
            pltpu.VMEM((N_DEV - 1, m_per, n_per), jnp.bfloat16),
            pltpu.VMEM((N_DEV - 1, m_per, n_per), jnp.bfloat16),
            pltpu.VMEM((m_per, n_per), jnp.float32),
            pltpu.SemaphoreType.DMA((2,)),
            pltpu.SemaphoreType.DMA((N_DEV - 1, 2)),
            pltpu.SemaphoreType.DMA((N_DEV - 1, 2)),
            pltpu.SemaphoreType.DMA((2,)),
        ],
        compiler_params=pltpu.CompilerParams(
            vmem_limit_bytes=62 * 1024 * 1024,
        ),
    )(x, w_mat)


# device time: 143799 ns/iter; 1.4864x vs baseline; 1.0679x over previous
import jax
import jax.numpy as jnp
from jax import lax
from jax.experimental import pallas as pl
from jax.experimental.pallas import tpu as pltpu

N_DEV = 4
NT = 4
SEQ = [(2, 0), (1, 0), (3, 0), (2, 1), (1, 1), (3, 1), (0, 0), (0, 1)]


def kernel(x, w_mat):
    m_per, k = x.shape
    _, n = w_mat.shape
    n_per = n // N_DEV
    tile = n_per // NT
    half = n_per // 2

    x = x.astype(jnp.bfloat16)

    def body(x_ref, w_ref, out_ref, w_buf, send_buf,
             w_sems, send_sems, recv_sems):
        my = lax.axis_index("i")

        steps = [(d, 2 * h + i) for (d, h) in SEQ for i in (0, 1)]

        def w_tile_copy(d, t, slot):
            dst = (my + d) % N_DEV
            return pltpu.make_async_copy(
                w_ref.at[:, pl.ds(dst * n_per + t * tile, tile)],
                w_buf.at[slot],
                w_sems.at[slot],
            )

        def remote_desc(d, h, rows):
            hs = pl.ds(h * half, half)
            return pltpu.make_async_remote_copy(
                src_ref=send_buf.at[d - 1, :, hs],
                dst_ref=out_ref.at[pl.ds(rows * m_per, m_per), hs],
                send_sem=send_sems.at[d - 1, h],
                recv_sem=recv_sems.at[d - 1, h],
                device_id=((my + d) % N_DEV,),
                device_id_type=pl.DeviceIdType.MESH,
            )

        w_tile_copy(*steps[0], 0).start()

        for s, (d, t) in enumerate(steps):
            slot = s % 2
            if s + 1 < len(steps):
                w_tile_copy(*steps[s + 1], (s + 1) % 2).start()
            w_tile_copy(d, t, slot).wait()

            wt = w_buf[slot].astype(jnp.bfloat16)
            yt = jnp.maximum(
                jnp.dot(x_ref[...], wt, preferred_element_type=jnp.float32),
                0.0,
            ).astype(jnp.bfloat16)
            if d == 0:
                out_ref[pl.ds(my * m_per, m_per), pl.ds(t * tile, tile)] = yt
            else:
                send_buf[d - 1, :, pl.ds(t * tile, tile)] = yt
                if t % 2 == 1:
                    remote_desc(d, t // 2, my).start()

        for d in [1, 3, 2]:
            src = (my - d) % N_DEV
            for h in (0, 1):
                remote_desc(d, h, src).wait_recv()
        for d in [1, 3, 2]:
            for h in (0, 1):
                remote_desc(d, h, my).wait_send()

    return pl.pallas_call(
        body,
        out_shape=jax.ShapeDtypeStruct((N_DEV * m_per, n_per), jnp.bfloat16),
        in_specs=[
            pl.BlockSpec(memory_space=pltpu.MemorySpace.VMEM),
            pl.BlockSpec(memory_space=pl.ANY),
        ],
        out_specs=pl.BlockSpec(memory_space=pltpu.MemorySpace.VMEM),
        scratch_shapes=[
            pltpu.VMEM((2, k, tile), jnp.float32),
            pltpu.VMEM((N_DEV - 1, m_per, n_per), jnp.bfloat16),
            pltpu.SemaphoreType.DMA((2,)),
            pltpu.SemaphoreType.DMA((N_DEV - 1, 2)),
            pltpu.SemaphoreType.DMA((N_DEV - 1, 2)),
        ],
        compiler_params=pltpu.CompilerParams(
            vmem_limit_bytes=62 * 1024 * 1024,
        ),
    )(x, w_mat)


# device time: 139634 ns/iter; 1.5307x vs baseline; 1.0298x over previous
import jax
import jax.numpy as jnp
from jax import lax
from jax.experimental import pallas as pl
from jax.experimental.pallas import tpu as pltpu

N_DEV = 4
NT = 4
SEQ = [(d, t) for t in range(NT) for d in (2, 1, 3)] + [(0, t) for t in range(NT)]


def kernel(x, w_mat):
    m_per, k = x.shape
    _, n = w_mat.shape
    n_per = n // N_DEV
    tile = n_per // NT

    x = x.astype(jnp.bfloat16)

    def body(x_ref, w_ref, out_ref, w_buf, send_buf,
             w_sems, send_sems, recv_sems):
        my = lax.axis_index("i")

        def w_tile_copy(d, t, slot):
            dst = (my + d) % N_DEV
            return pltpu.make_async_copy(
                w_ref.at[:, pl.ds(dst * n_per + t * tile, tile)],
                w_buf.at[slot],
                w_sems.at[slot],
            )

        def remote_desc(d, t, rows):
            ts = pl.ds(t * tile, tile)
            return pltpu.make_async_remote_copy(
                src_ref=send_buf.at[d - 1, :, ts],
                dst_ref=out_ref.at[pl.ds(rows * m_per, m_per), ts],
                send_sem=send_sems.at[d - 1, t],
                recv_sem=recv_sems.at[d - 1, t],
                device_id=((my + d) % N_DEV,),
                device_id_type=pl.DeviceIdType.MESH,
            )

        w_tile_copy(*SEQ[0], 0).start()

        for s, (d, t) in enumerate(SEQ):
            slot = s % 2
            if s + 1 < len(SEQ):
                w_tile_copy(*SEQ[s + 1], (s + 1) % 2).start()
            w_tile_copy(d, t, slot).wait()

            wt = w_buf[slot].astype(jnp.bfloat16)
            yt = jnp.maximum(
                jnp.dot(x_ref[...], wt, preferred_element_type=jnp.float32),
                0.0,
            ).astype(jnp.bfloat16)
            if d == 0:
                out_ref[pl.ds(my * m_per, m_per), pl.ds(t * tile, tile)] = yt
            else:
                send_buf[d - 1, :, pl.ds(t * tile, tile)] = yt
                remote_desc(d, t, my).start()

        for d in [1, 3, 2]:
            src = (my - d) % N_DEV
            for t in range(NT):
                remote_desc(d, t, src).wait_recv()
        for d in [1, 3, 2]:
            for t in range(NT):
                remote_desc(d, t, my).wait_send()

    return pl.pallas_call(
        body,
        out_shape=jax.ShapeDtypeStruct((N_DEV * m_per, n_per), jnp.bfloat16),
        in_specs=[
            pl.BlockSpec(memory_space=pltpu.MemorySpace.VMEM),
            pl.BlockSpec(memory_space=pl.ANY),
        ],
        out_specs=pl.BlockSpec(memory_space=pltpu.MemorySpace.VMEM),
        scratch_shapes=[
            pltpu.VMEM((2, k, tile), jnp.float32),
            pltpu.VMEM((N_DEV - 1, m_per, n_per), jnp.bfloat16),
            pltpu.SemaphoreType.DMA((2,)),
            pltpu.SemaphoreType.DMA((N_DEV - 1, NT)),
            pltpu.SemaphoreType.DMA((N_DEV - 1, NT)),
        ],
        compiler_params=pltpu.CompilerParams(
            vmem_limit_bytes=62 * 1024 * 1024,
        ),
    )(x, w_mat)
